# Initial kernel scaffold; baseline (speedup 1.0000x reference)
import jax
import jax.numpy as jnp
from jax.experimental import pallas as pl


def kernel(x, W1, b1, W2, b2, edge_index, rootindex, batch):
    raise NotImplementedError("write your pallas kernel here")


# speedup vs baseline: 12.1393x; 12.1393x over previous
import jax
import jax.numpy as jnp
from jax import lax
from jax.experimental import pallas as pl
from jax.experimental.pallas import tpu as pltpu
from jax.experimental.pallas import tpu_sc as plsc

N = 10000
E = 320000
FIN = 128
HID = 256
FOUT = 128
G = 128
NP_ = 10240
RB = 256
NBLK = NP_ // RB
EB = 128
NW = 32
BPW = 79
EPAD = NW * BPW * EB
RPT = NP_ // 16

_mesh = plsc.VectorSubcoreMesh(core_axis_name="c", subcore_axis_name="s")


def _sc_deg_body(dst_hbm, root_hbm, x_hbm, zero_hbm, ones_hbm,
                 deg_out, xroot_out,
                 acc_sh, idx, onesv, ridx, xr, sem):
    c = lax.axis_index("c")
    s = lax.axis_index("s")
    wid = s * 2 + c
    r0 = s * RPT
    pltpu.sync_copy(zero_hbm.at[pl.ds(r0, RPT)], acc_sh.at[pl.ds(r0, RPT)])
    pltpu.sync_copy(ones_hbm, onesv)

    @pl.when(wid == 0)
    def _():
        pltpu.sync_copy(root_hbm, ridx)
        pltpu.async_copy(x_hbm.at[ridx], xr, sem).wait()
        pltpu.sync_copy(xr, xroot_out)

    plsc.subcore_barrier()

    def body(i, carry):
        base = (wid * BPW + i) * EB
        pltpu.sync_copy(dst_hbm.at[pl.ds(base, EB)], idx)
        pltpu.sync_copy(onesv, acc_sh.at[idx], add=True)
        return carry

    lax.fori_loop(0, BPW, body, 0)
    plsc.subcore_barrier()
    pltpu.sync_copy(acc_sh.at[pl.ds(r0, RPT)],
                    deg_out.at[c, pl.ds(r0, RPT)])


_sc_deg = pl.kernel(
    _sc_deg_body,
    out_type=(jax.ShapeDtypeStruct((2, NP_), jnp.float32),
              jax.ShapeDtypeStruct((G, FIN), jnp.float32)),
    mesh=_mesh,
    scratch_types=[
        pltpu.VMEM_SHARED((NP_,), jnp.float32),
        pltpu.VMEM((EB,), jnp.int32),
        pltpu.VMEM((EB,), jnp.float32),
        pltpu.VMEM((G,), jnp.int32),
        pltpu.VMEM((G, FIN), jnp.float32),
        pltpu.SemaphoreType.DMA,
    ],
)


def _sc_agg1_body(src_hbm, dst_hbm, tab_hbm, zero_hbm,
                  parts_out,
                  acc_sh, sidx, didx, rows, sem):
    c = lax.axis_index("c")
    s = lax.axis_index("s")
    wid = s * 2 + c
    r0 = s * RPT
    pltpu.sync_copy(zero_hbm.at[pl.ds(r0, RPT)], acc_sh.at[pl.ds(r0, RPT)])
    plsc.subcore_barrier()

    def body(i, carry):
        base = (wid * BPW + i) * EB
        pltpu.sync_copy(src_hbm.at[pl.ds(base, EB)], sidx)
        pltpu.async_copy(tab_hbm.at[sidx], rows, sem).wait()
        pltpu.sync_copy(dst_hbm.at[pl.ds(base, EB)], didx)
        pltpu.sync_copy(rows, acc_sh.at[didx], add=True)
        return carry

    lax.fori_loop(0, BPW, body, 0)
    plsc.subcore_barrier()
    pltpu.sync_copy(acc_sh.at[pl.ds(r0, RPT)],
                    parts_out.at[c, pl.ds(r0, RPT)])


_sc_agg1 = pl.kernel(
    _sc_agg1_body,
    out_type=jax.ShapeDtypeStruct((2, NP_, FIN), jnp.float32),
    mesh=_mesh,
    scratch_types=[
        pltpu.VMEM_SHARED((NP_, FIN), jnp.float32),
        pltpu.VMEM((EB,), jnp.int32),
        pltpu.VMEM((EB,), jnp.int32),
        pltpu.VMEM((EB, FIN), jnp.float32),
        pltpu.SemaphoreType.DMA,
    ],
)


def _tc_prep_body(dp_ref, x_ref, xs_ref, disb_ref):
    d = jnp.sum(dp_ref[...], axis=0) + 1.0
    disv = lax.rsqrt(d)
    disb = jnp.broadcast_to(disv[:, None], (RB, FIN))
    disb_ref[...] = disb
    xs_ref[...] = disb * x_ref[...]


_tc_prep = pl.pallas_call(
    _tc_prep_body,
    grid=(NBLK,),
    in_specs=[pl.BlockSpec((2, RB), lambda i: (0, i)),
              pl.BlockSpec((RB, FIN), lambda i: (i, 0))],
    out_specs=[pl.BlockSpec((RB, FIN), lambda i: (i, 0)),
               pl.BlockSpec((RB, FIN), lambda i: (i, 0))],
    out_shape=[jax.ShapeDtypeStruct((NP_, FIN), jnp.float32),
               jax.ShapeDtypeStruct((NP_, FIN), jnp.float32)],
)


def _tc_mm1_body(p0_ref, p1_ref, xs_ref, disb_ref, w1_ref, b1_ref, h_ref):
    t = disb_ref[...] * (p0_ref[...] + p1_ref[...] + xs_ref[...])
    h_ref[...] = (jnp.dot(t, w1_ref[...], preferred_element_type=jnp.float32)
                  + b1_ref[...])


_tc_mm1 = pl.pallas_call(
    _tc_mm1_body,
    grid=(NBLK,),
    in_specs=[pl.BlockSpec((RB, FIN), lambda i: (i, 0)),
              pl.BlockSpec((RB, FIN), lambda i: (i, 0)),
              pl.BlockSpec((RB, FIN), lambda i: (i, 0)),
              pl.BlockSpec((RB, FIN), lambda i: (i, 0)),
              pl.BlockSpec((FIN, HID), lambda i: (0, 0)),
              pl.BlockSpec((1, HID), lambda i: (0, 0))],
    out_specs=pl.BlockSpec((RB, HID), lambda i: (i, 0)),
    out_shape=jax.ShapeDtypeStruct((NP_, HID), jnp.float32),
)


def _tc_r_body(xr_ref, w2b_ref, r_ref):
    r_ref[...] = jnp.dot(jnp.maximum(xr_ref[...], 0.0), w2b_ref[...],
                         preferred_element_type=jnp.float32)


_tc_r = pl.pallas_call(
    _tc_r_body,
    out_shape=jax.ShapeDtypeStruct((G, FOUT), jnp.float32),
)


def _tc_mm2_body(h_ref, w2a_ref, r_ref, b3_ref, disb_ref, z_ref):
    i = pl.program_id(0)
    zb = jnp.dot(jnp.maximum(h_ref[...], 0.0), w2a_ref[...],
                 preferred_element_type=jnp.float32)
    bv = b3_ref[...][0, 0]
    gid = lax.broadcasted_iota(jnp.int32, (RB, G), 1)
    oh = (bv[:, None] == gid).astype(jnp.float32)
    rb = jnp.dot(oh, r_ref[...], preferred_element_type=jnp.float32)
    rowid = i * RB + lax.broadcasted_iota(jnp.int32, (RB, FOUT), 0)
    z = disb_ref[...] * (zb + rb)
    z_ref[...] = jnp.where(rowid < N, z, 0.0)


_tc_mm2 = pl.pallas_call(
    _tc_mm2_body,
    grid=(NBLK,),
    in_specs=[pl.BlockSpec((RB, HID), lambda i: (i, 0)),
              pl.BlockSpec((HID, FOUT), lambda i: (0, 0)),
              pl.BlockSpec((G, FOUT), lambda i: (0, 0)),
              pl.BlockSpec((1, 1, RB), lambda i: (i, 0, 0)),
              pl.BlockSpec((RB, FIN), lambda i: (i, 0))],
    out_specs=pl.BlockSpec((RB, FOUT), lambda i: (i, 0)),
    out_shape=jax.ShapeDtypeStruct((NP_, FOUT), jnp.float32),
)


def _tc_final_body(q0_ref, q1_ref, z_ref, disb_ref, b2_ref, b3_ref, h_ref,
                   root3_ref, out_ref, sacc, cacc, hracc):
    i = pl.program_id(0)
    w = jnp.maximum(
        disb_ref[...] * (q0_ref[...] + q1_ref[...] + z_ref[...]) + b2_ref[...],
        0.0)
    bv = b3_ref[...][0, 0]
    gid = lax.broadcasted_iota(jnp.int32, (RB, G), 1)
    oh = (bv[:, None] == gid).astype(jnp.float32)
    dnum = (((0,), (0,)), ((), ()))
    scon = lax.dot_general(oh, w, dnum, preferred_element_type=jnp.float32)
    ccon = lax.dot_general(oh, jnp.ones((RB, FOUT), jnp.float32), dnum,
                           preferred_element_type=jnp.float32)
    rv = root3_ref[...][0, 0]
    rowid = i * RB + lax.broadcasted_iota(jnp.int32, (RB, G), 0)
    ohr = (rowid == rv[None, :]).astype(jnp.float32)
    rcon = lax.dot_general(ohr, h_ref[...], dnum,
                           preferred_element_type=jnp.float32)

    @pl.when(i == 0)
    def _():
        sacc[...] = scon
        cacc[...] = ccon
        hracc[...] = rcon

    @pl.when(i > 0)
    def _():
        sacc[...] += scon
        cacc[...] += ccon
        hracc[...] += rcon

    @pl.when(i == NBLK - 1)
    def _():
        cnt = cacc[...]
        pooled = sacc[...] / jnp.maximum(cnt, 1.0)
        alive = jnp.broadcast_to((cnt[:, :1] > 0.0), (G, HID)).astype(
            jnp.float32)
        out_ref[...] = jnp.concatenate([pooled, hracc[...] * alive],
                                       axis=1)


_tc_final = pl.pallas_call(
    _tc_final_body,
    grid=(NBLK,),
    in_specs=[pl.BlockSpec((RB, FOUT), lambda i: (i, 0)),
              pl.BlockSpec((RB, FOUT), lambda i: (i, 0)),
              pl.BlockSpec((RB, FOUT), lambda i: (i, 0)),
              pl.BlockSpec((RB, FIN), lambda i: (i, 0)),
              pl.BlockSpec((1, FOUT), lambda i: (0, 0)),
              pl.BlockSpec((1, 1, RB), lambda i: (i, 0, 0)),
              pl.BlockSpec((RB, HID), lambda i: (i, 0)),
              pl.BlockSpec((1, 1, G), lambda i: (0, 0, 0))],
    out_specs=pl.BlockSpec((G, FOUT + HID), lambda i: (0, 0)),
    out_shape=jax.ShapeDtypeStruct((G, FOUT + HID), jnp.float32),
    scratch_shapes=[pltpu.VMEM((G, FOUT), jnp.float32),
                    pltpu.VMEM((G, FOUT), jnp.float32),
                    pltpu.VMEM((G, HID), jnp.float32)],
)


def kernel(x, W1, b1, W2, b2, edge_index, rootindex, batch):
    x = x.astype(jnp.float32)
    src = edge_index[0].astype(jnp.int32)
    dst = edge_index[1].astype(jnp.int32)
    pad_e = jnp.full((EPAD - E,), N, jnp.int32)
    srcp = jnp.concatenate([src, pad_e])
    dstp = jnp.concatenate([dst, pad_e])
    xpad = jnp.pad(x, ((0, NP_ - N), (0, 0)))
    batchp = jnp.pad(batch.astype(jnp.int32), (0, NP_ - N),
                     constant_values=G).reshape(NBLK, 1, RB)
    zeros1 = jnp.zeros((NP_,), jnp.float32)
    zeros2 = jnp.zeros((NP_, FIN), jnp.float32)
    onese = jnp.ones((EB,), jnp.float32)
    root = rootindex.astype(jnp.int32)
    W2a = W2[:HID]
    W2b = W2[HID:]
    b1r = b1.reshape(1, HID)
    b2r = b2.reshape(1, FOUT)

    deg_parts, xroot = _sc_deg(dstp, root, xpad, zeros1, onese)
    xs, disb = _tc_prep(deg_parts, xpad)
    parts1 = _sc_agg1(srcp, dstp, xs, zeros2)
    h = _tc_mm1(parts1[0], parts1[1], xs, disb, W1, b1r)
    r = _tc_r(xroot, W2b)
    z = _tc_mm2(h, W2a, r, batchp, disb)
    parts2 = _sc_agg1(srcp, dstp, z, zeros2)
    root3 = root.reshape(1, 1, G)
    out = _tc_final(parts2[0], parts2[1], z, disb, b2r, batchp, h, root3)
    return out
